# baseline (speedup 1.0000x reference)
import functools

import jax
import jax.numpy as jnp
from jax.experimental import pallas as pl

_INTERPRET = False

EB = 1000
NB = 1000


def _mm(a, b):
    return jax.lax.dot_general(
        a, b, (((1,), (0,)), ((), ())), preferred_element_type=jnp.float32
    )


def _pcall(body, grid, in_specs, out_specs, out_shape):
    return pl.pallas_call(
        body,
        grid=grid,
        in_specs=in_specs,
        out_specs=out_specs,
        out_shape=out_shape,
        interpret=_INTERPRET,
    )


def _full(shape):
    return pl.BlockSpec(shape, lambda i: tuple(0 for _ in shape))


def _mlp1_body(ea_ref, w0_ref, b0_ref, w1_ref, b1_ref, w2_ref, b2_ref, o_ref):
    h = jnp.maximum(ea_ref[...] * w0_ref[...] + b0_ref[...], 0.0)
    h = jnp.maximum(_mm(h, w1_ref[...]) + b1_ref[...], 0.0)
    o_ref[...] = _mm(h, w2_ref[...]) + b2_ref[...]


def _mlp1(ea, p, block):
    e = ea.shape[0]
    dout = p["W2"].shape[1]
    return _pcall(
        _mlp1_body,
        grid=(e // block,),
        in_specs=[
            pl.BlockSpec((block, 1), lambda i: (i, 0)),
            _full((1, 128)), _full((1, 128)),
            _full((128, 128)), _full((1, 128)),
            _full((128, dout)), _full((1, dout)),
        ],
        out_specs=pl.BlockSpec((block, dout), lambda i: (i, 0)),
        out_shape=jax.ShapeDtypeStruct((e, dout), jnp.float32),
    )(ea, p["W0"].reshape(1, 128), p["b0"].reshape(1, 128),
      p["W1"], p["b1"].reshape(1, 128), p["W2"], p["b2"].reshape(1, -1))


def _proj_body(x_ref, la_ref, wl_ref, bl_ref, wr_ref, br_ref, we_ref, att_ref,
               xl_ref, xr_ref, al_ref):
    x = x_ref[...]
    xl = _mm(x, wl_ref[...]) + bl_ref[...]
    xr = _mm(x, wr_ref[...]) + br_ref[...]
    s = xl + xr + _mm(la_ref[...], we_ref[...])
    s = jnp.where(s > 0, s, 0.2 * s)
    xl_ref[...] = xl
    xr_ref[...] = xr
    al_ref[...] = jnp.sum(s * att_ref[...], axis=1, keepdims=True)


def _proj(x, la, p):
    n, c = x.shape
    out_shape = (
        jax.ShapeDtypeStruct((n, 128), jnp.float32),
        jax.ShapeDtypeStruct((n, 128), jnp.float32),
        jax.ShapeDtypeStruct((n, 1), jnp.float32),
    )
    return _pcall(
        _proj_body,
        grid=(n // NB,),
        in_specs=[
            pl.BlockSpec((NB, c), lambda i: (i, 0)),
            pl.BlockSpec((NB, 128), lambda i: (i, 0)),
            _full((c, 128)), _full((1, 128)),
            _full((c, 128)), _full((1, 128)),
            _full((128, 128)), _full((1, 128)),
        ],
        out_specs=(
            pl.BlockSpec((NB, 128), lambda i: (i, 0)),
            pl.BlockSpec((NB, 128), lambda i: (i, 0)),
            pl.BlockSpec((NB, 1), lambda i: (i, 0)),
        ),
        out_shape=out_shape,
    )(x, la, p["Wl"], p["bl"].reshape(1, 128), p["Wr"], p["br"].reshape(1, 128),
      p["We"], p["att"].reshape(1, 128))


def _alpha_body(gxl_ref, gxr_ref, e_ref, we_ref, att_ref, a_ref):
    s = gxl_ref[...] + gxr_ref[...] + _mm(e_ref[...], we_ref[...])
    s = jnp.where(s > 0, s, 0.2 * s)
    a_ref[...] = jnp.sum(s * att_ref[...], axis=1, keepdims=True)


def _alpha(gxl, gxr, e, we, att):
    ne = gxl.shape[0]
    return _pcall(
        _alpha_body,
        grid=(ne // EB,),
        in_specs=[
            pl.BlockSpec((EB, 128), lambda i: (i, 0)),
            pl.BlockSpec((EB, 128), lambda i: (i, 0)),
            pl.BlockSpec((EB, 128), lambda i: (i, 0)),
            _full((128, 128)), _full((1, 128)),
        ],
        out_specs=pl.BlockSpec((EB, 1), lambda i: (i, 0)),
        out_shape=jax.ShapeDtypeStruct((ne, 1), jnp.float32),
    )(gxl, gxr, e, we, att.reshape(1, 128))


def _weight_body(gxl_ref, a_ref, am_ref, w_ref):
    ex = jnp.exp(a_ref[...] - am_ref[...])
    w_ref[:, :128] = gxl_ref[...] * ex
    w_ref[:, 128:] = ex


def _weight(gxl, alpha, amax_g):
    ne = gxl.shape[0]
    return _pcall(
        _weight_body,
        grid=(ne // EB,),
        in_specs=[
            pl.BlockSpec((EB, 128), lambda i: (i, 0)),
            pl.BlockSpec((EB, 1), lambda i: (i, 0)),
            pl.BlockSpec((EB, 1), lambda i: (i, 0)),
        ],
        out_specs=pl.BlockSpec((EB, 129), lambda i: (i, 0)),
        out_shape=jax.ShapeDtypeStruct((ne, 129), jnp.float32),
    )(gxl, alpha, amax_g)


def _nmm_body(x_ref, w_ref, o_ref):
    o_ref[...] = _mm(x_ref[...], w_ref[...])


def _nmm(x, w):
    n, c = x.shape
    dout = w.shape[1]
    return _pcall(
        _nmm_body,
        grid=(n // NB,),
        in_specs=[pl.BlockSpec((NB, c), lambda i: (i, 0)), _full((c, dout))],
        out_specs=pl.BlockSpec((NB, dout), lambda i: (i, 0)),
        out_shape=jax.ShapeDtypeStruct((n, dout), jnp.float32),
    )(x, w)


def _dec_body(ps_ref, pd_ref, e_ref, w0c_ref, b0_ref, w1_ref, b1_ref,
              w2_ref, b2_ref, o_ref):
    h = ps_ref[...] + pd_ref[...] + _mm(e_ref[...], w0c_ref[...]) + b0_ref[...]
    h = jnp.maximum(h, 0.0)
    h = jnp.maximum(_mm(h, w1_ref[...]) + b1_ref[...], 0.0)
    o_ref[...] = _mm(h, w2_ref[...]) + b2_ref[...]


def _decoder(ps_g, pd_g, e1, p):
    ne = ps_g.shape[0]
    return _pcall(
        _dec_body,
        grid=(ne // EB,),
        in_specs=[
            pl.BlockSpec((EB, 128), lambda i: (i, 0)),
            pl.BlockSpec((EB, 128), lambda i: (i, 0)),
            pl.BlockSpec((EB, 128), lambda i: (i, 0)),
            _full((128, 128)), _full((1, 128)),
            _full((128, 128)), _full((1, 128)),
            _full((128, 1)), _full((1, 1)),
        ],
        out_specs=pl.BlockSpec((EB, 1), lambda i: (i, 0)),
        out_shape=jax.ShapeDtypeStruct((ne, 1), jnp.float32),
    )(ps_g, pd_g, e1, p["W0"][512:], p["b0"].reshape(1, 128),
      p["W1"], p["b1"].reshape(1, 128), p["W2"], p["b2"].reshape(1, 1))


def _gat_layer(x, src, dst, e, la, p, n):
    xl, xr, aloop = _proj(x, la, p)
    gxl = jnp.take(xl, src, axis=0)
    gxr = jnp.take(xr, dst, axis=0)
    alpha = _alpha(gxl, gxr, e, p["We"], p["att"])
    af = alpha[:, 0]
    amax = jnp.maximum(
        jax.ops.segment_max(af, dst, num_segments=n), aloop[:, 0]
    )
    wex = _weight(gxl, alpha, jnp.take(amax, dst)[:, None])
    sboth = jax.ops.segment_sum(wex, dst, num_segments=n)
    ssum, dsum = sboth[:, :128], sboth[:, 128]
    exl = jnp.exp(aloop[:, 0] - amax)
    denom = dsum + exl + 1e-16
    return (ssum + exl[:, None] * xl) / denom[:, None] + p["bias"]


def _loop_attr(e, dst, n):
    e1s = jnp.concatenate([e, jnp.ones((e.shape[0], 1), jnp.float32)], axis=1)
    s = jax.ops.segment_sum(e1s, dst, num_segments=n)
    return s[:, :128] / jnp.maximum(s[:, 128], 1.0)[:, None]


def kernel(graph1_x, graph1_edge_index, graph1_edge_attr, graph1_batch,
           graph2_x, graph2_edge_index, graph2_edge_attr, t_value, params):
    n = graph1_x.shape[0]
    src1, dst1 = graph1_edge_index[0], graph1_edge_index[1]
    src2, dst2 = graph2_edge_index[0], graph2_edge_index[1]

    te = _mlp1(t_value[:, None], params["time_encoder"], t_value.shape[0])
    x = jnp.take(jnp.concatenate([te, te], axis=1), graph1_batch, axis=0)

    e1 = _mlp1(graph1_edge_attr[:, 0:1], params["encoder_edges"], EB)
    e2 = _mlp1(graph2_edge_attr[:, None], params["encoder_edges"], EB)

    la1 = _loop_attr(e1, dst1, n)
    la2 = _loop_attr(e2, dst2, n)

    for i in range(3):
        o1 = _gat_layer(x, src1, dst1, e1, la1, params["gnn_global"][i], n)
        o2 = _gat_layer(x, src2, dst2, e2, la2, params["gnn_filter"][i], n)
        x = jnp.concatenate([o1, o2], axis=1)

    dp = params["decoding_layer_edge"]
    ps = _nmm(x, dp["W0"][:256])
    pd = _nmm(x, dp["W0"][256:512])
    return _decoder(jnp.take(ps, src1, axis=0), jnp.take(pd, dst1, axis=0),
                    e1, dp)


# speedup vs baseline: 1.1081x; 1.1081x over previous
import functools

import jax
import jax.numpy as jnp
from jax import lax
from jax.experimental import pallas as pl
from jax.experimental.pallas import tpu as pltpu
from jax.experimental.pallas import tpu_sc as plsc

_INTERPRET = False

EB = 1000
NB = 1000


def _mm(a, b):
    return jax.lax.dot_general(
        a, b, (((1,), (0,)), ((), ())), preferred_element_type=jnp.float32
    )


def _pcall(body, grid, in_specs, out_specs, out_shape):
    return pl.pallas_call(
        body,
        grid=grid,
        in_specs=in_specs,
        out_specs=out_specs,
        out_shape=out_shape,
        interpret=_INTERPRET,
    )


def _full(shape):
    return pl.BlockSpec(shape, lambda i: tuple(0 for _ in shape))


def _mlp1_body(ea_ref, w0_ref, b0_ref, w1_ref, b1_ref, w2_ref, b2_ref, o_ref):
    h = jnp.maximum(ea_ref[...] * w0_ref[...] + b0_ref[...], 0.0)
    h = jnp.maximum(_mm(h, w1_ref[...]) + b1_ref[...], 0.0)
    o_ref[...] = _mm(h, w2_ref[...]) + b2_ref[...]


def _mlp1(ea, p, block):
    e = ea.shape[0]
    dout = p["W2"].shape[1]
    return _pcall(
        _mlp1_body,
        grid=(e // block,),
        in_specs=[
            pl.BlockSpec((block, 1), lambda i: (i, 0)),
            _full((1, 128)), _full((1, 128)),
            _full((128, 128)), _full((1, 128)),
            _full((128, dout)), _full((1, dout)),
        ],
        out_specs=pl.BlockSpec((block, dout), lambda i: (i, 0)),
        out_shape=jax.ShapeDtypeStruct((e, dout), jnp.float32),
    )(ea, p["W0"].reshape(1, 128), p["b0"].reshape(1, 128),
      p["W1"], p["b1"].reshape(1, 128), p["W2"], p["b2"].reshape(1, -1))


def _proj_body(x_ref, la_ref, wl_ref, bl_ref, wr_ref, br_ref, we_ref, att_ref,
               xl_ref, xr_ref, al_ref):
    x = x_ref[...]
    xl = _mm(x, wl_ref[...]) + bl_ref[...]
    xr = _mm(x, wr_ref[...]) + br_ref[...]
    s = xl + xr + _mm(la_ref[...], we_ref[...])
    s = jnp.where(s > 0, s, 0.2 * s)
    xl_ref[...] = xl
    xr_ref[...] = xr
    al_ref[...] = jnp.sum(s * att_ref[...], axis=1, keepdims=True)


def _proj(x, la, p):
    n, c = x.shape
    out_shape = (
        jax.ShapeDtypeStruct((n, 128), jnp.float32),
        jax.ShapeDtypeStruct((n, 128), jnp.float32),
        jax.ShapeDtypeStruct((n, 1), jnp.float32),
    )
    return _pcall(
        _proj_body,
        grid=(n // NB,),
        in_specs=[
            pl.BlockSpec((NB, c), lambda i: (i, 0)),
            pl.BlockSpec((NB, 128), lambda i: (i, 0)),
            _full((c, 128)), _full((1, 128)),
            _full((c, 128)), _full((1, 128)),
            _full((128, 128)), _full((1, 128)),
        ],
        out_specs=(
            pl.BlockSpec((NB, 128), lambda i: (i, 0)),
            pl.BlockSpec((NB, 128), lambda i: (i, 0)),
            pl.BlockSpec((NB, 1), lambda i: (i, 0)),
        ),
        out_shape=out_shape,
    )(x, la, p["Wl"], p["bl"].reshape(1, 128), p["Wr"], p["br"].reshape(1, 128),
      p["We"], p["att"].reshape(1, 128))


def _alpha_body(gxl_ref, gxr_ref, e_ref, we_ref, att_ref, a_ref):
    s = gxl_ref[...] + gxr_ref[...] + _mm(e_ref[...], we_ref[...])
    s = jnp.where(s > 0, s, 0.2 * s)
    a_ref[...] = jnp.sum(s * att_ref[...], axis=1, keepdims=True)


def _alpha(gxl, gxr, e, we, att):
    ne = gxl.shape[0]
    return _pcall(
        _alpha_body,
        grid=(ne // EB,),
        in_specs=[
            pl.BlockSpec((EB, 128), lambda i: (i, 0)),
            pl.BlockSpec((EB, 128), lambda i: (i, 0)),
            pl.BlockSpec((EB, 128), lambda i: (i, 0)),
            _full((128, 128)), _full((1, 128)),
        ],
        out_specs=pl.BlockSpec((EB, 1), lambda i: (i, 0)),
        out_shape=jax.ShapeDtypeStruct((ne, 1), jnp.float32),
    )(gxl, gxr, e, we, att.reshape(1, 128))


def _weight_body(gxl_ref, a_ref, am_ref, w_ref, ex_ref):
    ex = jnp.exp(a_ref[...] - am_ref[...])
    ex_ref[...] = ex
    w_ref[...] = gxl_ref[...] * ex


def _weight(gxl, alpha, amax_g):
    ne = gxl.shape[0]
    return _pcall(
        _weight_body,
        grid=(ne // EB,),
        in_specs=[
            pl.BlockSpec((EB, 128), lambda i: (i, 0)),
            pl.BlockSpec((EB, 1), lambda i: (i, 0)),
            pl.BlockSpec((EB, 1), lambda i: (i, 0)),
        ],
        out_specs=(
            pl.BlockSpec((EB, 128), lambda i: (i, 0)),
            pl.BlockSpec((EB, 1), lambda i: (i, 0)),
        ),
        out_shape=(
            jax.ShapeDtypeStruct((ne, 128), jnp.float32),
            jax.ShapeDtypeStruct((ne, 1), jnp.float32),
        ),
    )(gxl, alpha, amax_g)


_SC_C = 128
_SC_K = 4
_SC_S = _SC_C * _SC_K
_SC_NW = 32


def _sc_pad(e):
    m = _SC_S * _SC_NW
    return ((e + m - 1) // m) * m


def _sc_gather_pair_body(ta_ref, ia_ref, tb_ref, ib_ref, oa_ref, ob_ref,
                         idx_v, rows_v, sem):
    wid = lax.axis_index("s") * 2 + lax.axis_index("c")
    per_w = ia_ref.shape[0] // _SC_NW

    def one(t_ref, i_ref, o_ref):
        nch = per_w // _SC_S

        def body(j, carry):
            off = wid * per_w + j * _SC_S
            pltpu.sync_copy(i_ref.at[pl.ds(off, _SC_S)], idx_v)
            handles = [
                pltpu.async_copy(
                    t_ref.at[idx_v.at[pl.ds(b * _SC_C, _SC_C)]],
                    rows_v.at[pl.ds(b * _SC_C, _SC_C)],
                    sem,
                )
                for b in range(_SC_K)
            ]
            for h in handles:
                h.wait()
            pltpu.sync_copy(rows_v, o_ref.at[pl.ds(off, _SC_S)])
            return carry

        lax.fori_loop(0, nch, body, 0)

    one(ta_ref, ia_ref, oa_ref)
    one(tb_ref, ib_ref, ob_ref)


def _sc_gather_pair(ta, ia, tb, ib):
    e = ia.shape[0]
    ep = _sc_pad(e)
    pad = ep - e
    ia_p = jnp.concatenate([ia, jnp.zeros((pad,), ia.dtype)])
    ib_p = jnp.concatenate([ib, jnp.zeros((pad,), ib.dtype)])
    run = pl.kernel(
        _sc_gather_pair_body,
        mesh=plsc.VectorSubcoreMesh(core_axis_name="c", subcore_axis_name="s"),
        out_type=(
            jax.ShapeDtypeStruct((ep, 128), jnp.float32),
            jax.ShapeDtypeStruct((ep, 128), jnp.float32),
        ),
        scratch_types=[
            pltpu.VMEM((_SC_S,), jnp.int32),
            pltpu.VMEM((_SC_S, 128), jnp.float32),
            pltpu.SemaphoreType.DMA,
        ],
    )
    ga, gb = run(ta, ia_p, tb, ib_p)
    return ga[:e], gb[:e]


def _nmm_body(x_ref, w_ref, o_ref):
    o_ref[...] = _mm(x_ref[...], w_ref[...])


def _nmm(x, w):
    n, c = x.shape
    dout = w.shape[1]
    return _pcall(
        _nmm_body,
        grid=(n // NB,),
        in_specs=[pl.BlockSpec((NB, c), lambda i: (i, 0)), _full((c, dout))],
        out_specs=pl.BlockSpec((NB, dout), lambda i: (i, 0)),
        out_shape=jax.ShapeDtypeStruct((n, dout), jnp.float32),
    )(x, w)


def _dec_body(ps_ref, pd_ref, e_ref, w0c_ref, b0_ref, w1_ref, b1_ref,
              w2_ref, b2_ref, o_ref):
    h = ps_ref[...] + pd_ref[...] + _mm(e_ref[...], w0c_ref[...]) + b0_ref[...]
    h = jnp.maximum(h, 0.0)
    h = jnp.maximum(_mm(h, w1_ref[...]) + b1_ref[...], 0.0)
    o_ref[...] = _mm(h, w2_ref[...]) + b2_ref[...]


def _decoder(ps_g, pd_g, e1, p):
    ne = ps_g.shape[0]
    return _pcall(
        _dec_body,
        grid=(ne // EB,),
        in_specs=[
            pl.BlockSpec((EB, 128), lambda i: (i, 0)),
            pl.BlockSpec((EB, 128), lambda i: (i, 0)),
            pl.BlockSpec((EB, 128), lambda i: (i, 0)),
            _full((128, 128)), _full((1, 128)),
            _full((128, 128)), _full((1, 128)),
            _full((128, 1)), _full((1, 1)),
        ],
        out_specs=pl.BlockSpec((EB, 1), lambda i: (i, 0)),
        out_shape=jax.ShapeDtypeStruct((ne, 1), jnp.float32),
    )(ps_g, pd_g, e1, p["W0"][512:], p["b0"].reshape(1, 128),
      p["W1"], p["b1"].reshape(1, 128), p["W2"], p["b2"].reshape(1, 1))


def _gat_layer(x, src, dst, e, la, p, n):
    xl, xr, aloop = _proj(x, la, p)
    gxl, gxr = _sc_gather_pair(xl, src, xr, dst)
    alpha = _alpha(gxl, gxr, e, p["We"], p["att"])
    af = alpha[:, 0]
    amax = jnp.maximum(
        jax.ops.segment_max(af, dst, num_segments=n), aloop[:, 0]
    )
    w, ex = _weight(gxl, alpha, jnp.take(amax, dst)[:, None])
    ssum = jax.ops.segment_sum(w, dst, num_segments=n)
    dsum = jax.ops.segment_sum(ex[:, 0], dst, num_segments=n)
    exl = jnp.exp(aloop[:, 0] - amax)
    denom = dsum + exl + 1e-16
    return (ssum + exl[:, None] * xl) / denom[:, None] + p["bias"]


def _loop_attr(e, dst, n):
    ea_sum = jax.ops.segment_sum(e, dst, num_segments=n)
    cnt = jax.ops.segment_sum(
        jnp.ones((e.shape[0],), jnp.float32), dst, num_segments=n
    )
    return ea_sum / jnp.maximum(cnt, 1.0)[:, None]


def kernel(graph1_x, graph1_edge_index, graph1_edge_attr, graph1_batch,
           graph2_x, graph2_edge_index, graph2_edge_attr, t_value, params):
    n = graph1_x.shape[0]
    src1, dst1 = graph1_edge_index[0], graph1_edge_index[1]
    src2, dst2 = graph2_edge_index[0], graph2_edge_index[1]

    te = _mlp1(t_value[:, None], params["time_encoder"], t_value.shape[0])
    x = jnp.take(jnp.concatenate([te, te], axis=1), graph1_batch, axis=0)

    e1 = _mlp1(graph1_edge_attr[:, 0:1], params["encoder_edges"], EB)
    e2 = _mlp1(graph2_edge_attr[:, None], params["encoder_edges"], EB)

    la1 = _loop_attr(e1, dst1, n)
    la2 = _loop_attr(e2, dst2, n)

    for i in range(3):
        o1 = _gat_layer(x, src1, dst1, e1, la1, params["gnn_global"][i], n)
        o2 = _gat_layer(x, src2, dst2, e2, la2, params["gnn_filter"][i], n)
        x = jnp.concatenate([o1, o2], axis=1)

    dp = params["decoding_layer_edge"]
    ps = _nmm(x, dp["W0"][:256])
    pd = _nmm(x, dp["W0"][256:512])
    gps, gpd = _sc_gather_pair(ps, src1, pd, dst1)
    return _decoder(gps, gpd, e1, dp)


# speedup vs baseline: 1.1348x; 1.0240x over previous
import functools

import jax
import jax.numpy as jnp
from jax import lax
from jax.experimental import pallas as pl
from jax.experimental.pallas import tpu as pltpu
from jax.experimental.pallas import tpu_sc as plsc

_INTERPRET = False

EB = 1000
NB = 1000


def _mm(a, b):
    return jax.lax.dot_general(
        a, b, (((1,), (0,)), ((), ())), preferred_element_type=jnp.float32
    )


def _pcall(body, grid, in_specs, out_specs, out_shape):
    return pl.pallas_call(
        body,
        grid=grid,
        in_specs=in_specs,
        out_specs=out_specs,
        out_shape=out_shape,
        interpret=_INTERPRET,
    )


def _full(shape):
    return pl.BlockSpec(shape, lambda i: tuple(0 for _ in shape))


def _mlp1_body(ea_ref, w0_ref, b0_ref, w1_ref, b1_ref, w2_ref, b2_ref, o_ref):
    h = jnp.maximum(ea_ref[...] * w0_ref[...] + b0_ref[...], 0.0)
    h = jnp.maximum(_mm(h, w1_ref[...]) + b1_ref[...], 0.0)
    o_ref[...] = _mm(h, w2_ref[...]) + b2_ref[...]


def _mlp1(ea, p, block):
    e = ea.shape[0]
    dout = p["W2"].shape[1]
    return _pcall(
        _mlp1_body,
        grid=(e // block,),
        in_specs=[
            pl.BlockSpec((block, 1), lambda i: (i, 0)),
            _full((1, 128)), _full((1, 128)),
            _full((128, 128)), _full((1, 128)),
            _full((128, dout)), _full((1, dout)),
        ],
        out_specs=pl.BlockSpec((block, dout), lambda i: (i, 0)),
        out_shape=jax.ShapeDtypeStruct((e, dout), jnp.float32),
    )(ea, p["W0"].reshape(1, 128), p["b0"].reshape(1, 128),
      p["W1"], p["b1"].reshape(1, 128), p["W2"], p["b2"].reshape(1, -1))


def _proj_body(x_ref, la_ref, wl_ref, bl_ref, wr_ref, br_ref, we_ref, att_ref,
               xl_ref, xr_ref, al_ref):
    x = x_ref[...]
    xl = _mm(x, wl_ref[...]) + bl_ref[...]
    xr = _mm(x, wr_ref[...]) + br_ref[...]
    s = xl + xr + _mm(la_ref[...], we_ref[...])
    s = jnp.where(s > 0, s, 0.2 * s)
    xl_ref[...] = xl
    xr_ref[...] = xr
    al_ref[...] = jnp.sum(s * att_ref[...], axis=1, keepdims=True)


def _proj(x, la, p):
    n, c = x.shape
    out_shape = (
        jax.ShapeDtypeStruct((n, 128), jnp.float32),
        jax.ShapeDtypeStruct((n, 128), jnp.float32),
        jax.ShapeDtypeStruct((n, 1), jnp.float32),
    )
    return _pcall(
        _proj_body,
        grid=(n // NB,),
        in_specs=[
            pl.BlockSpec((NB, c), lambda i: (i, 0)),
            pl.BlockSpec((NB, 128), lambda i: (i, 0)),
            _full((c, 128)), _full((1, 128)),
            _full((c, 128)), _full((1, 128)),
            _full((128, 128)), _full((1, 128)),
        ],
        out_specs=(
            pl.BlockSpec((NB, 128), lambda i: (i, 0)),
            pl.BlockSpec((NB, 128), lambda i: (i, 0)),
            pl.BlockSpec((NB, 1), lambda i: (i, 0)),
        ),
        out_shape=out_shape,
    )(x, la, p["Wl"], p["bl"].reshape(1, 128), p["Wr"], p["br"].reshape(1, 128),
      p["We"], p["att"].reshape(1, 128))


def _alpha_body(gxl_ref, gxr_ref, e_ref, we_ref, att_ref, a_ref):
    s = gxl_ref[...] + gxr_ref[...] + _mm(e_ref[...], we_ref[...])
    s = jnp.where(s > 0, s, 0.2 * s)
    a_ref[...] = jnp.sum(s * att_ref[...], axis=1, keepdims=True)


def _alpha(gxl, gxr, e, we, att):
    ne = gxl.shape[0]
    return _pcall(
        _alpha_body,
        grid=(ne // EB,),
        in_specs=[
            pl.BlockSpec((EB, 128), lambda i: (i, 0)),
            pl.BlockSpec((EB, 128), lambda i: (i, 0)),
            pl.BlockSpec((EB, 128), lambda i: (i, 0)),
            _full((128, 128)), _full((1, 128)),
        ],
        out_specs=pl.BlockSpec((EB, 1), lambda i: (i, 0)),
        out_shape=jax.ShapeDtypeStruct((ne, 1), jnp.float32),
    )(gxl, gxr, e, we, att.reshape(1, 128))


def _weight_body(gxl_ref, a_ref, am_ref, w_ref, ex_ref):
    ex = jnp.exp(a_ref[...] - am_ref[...])
    ex_ref[...] = ex
    w_ref[...] = gxl_ref[...] * ex


def _weight(gxl, alpha, amax_g):
    ne = gxl.shape[0]
    return _pcall(
        _weight_body,
        grid=(ne // EB,),
        in_specs=[
            pl.BlockSpec((EB, 128), lambda i: (i, 0)),
            pl.BlockSpec((EB, 1), lambda i: (i, 0)),
            pl.BlockSpec((EB, 1), lambda i: (i, 0)),
        ],
        out_specs=(
            pl.BlockSpec((EB, 128), lambda i: (i, 0)),
            pl.BlockSpec((EB, 1), lambda i: (i, 0)),
        ),
        out_shape=(
            jax.ShapeDtypeStruct((ne, 128), jnp.float32),
            jax.ShapeDtypeStruct((ne, 1), jnp.float32),
        ),
    )(gxl, alpha, amax_g)


_SC_C = 128
_SC_K = 2
_SC_S = _SC_C * _SC_K
_SC_NW = 32


def _sc_pad(e):
    m = 2 * _SC_S * _SC_NW
    return ((e + m - 1) // m) * m


def _sc_gather_pair_body(ta_ref, ia_ref, tb_ref, ib_ref, oa_ref, ob_ref,
                         idx_a, idx_b, rows_a, rows_b, sem_a, sem_b):
    wid = lax.axis_index("s") * 2 + lax.axis_index("c")
    per_w = ia_ref.shape[0] // _SC_NW

    def one(t_ref, i_ref, o_ref):
        base = wid * per_w
        nhalf = per_w // (2 * _SC_S)

        def fire(iv, rv, sem):
            for b in range(_SC_K):
                pltpu.async_copy(
                    t_ref.at[iv.at[pl.ds(b * _SC_C, _SC_C)]],
                    rv.at[pl.ds(b * _SC_C, _SC_C)],
                    sem,
                )

        def drain(rv, sem):
            pltpu.make_async_copy(t_ref.at[pl.ds(0, _SC_S)], rv, sem).wait()

        def stage(chunk, iv, rv, sem):
            pltpu.sync_copy(i_ref.at[pl.ds(base + chunk * _SC_S, _SC_S)], iv)
            fire(iv, rv, sem)

        stage(0, idx_a, rows_a, sem_a)

        def body(k, carry):
            c0 = 2 * k
            stage(c0 + 1, idx_b, rows_b, sem_b)
            drain(rows_a, sem_a)
            pltpu.sync_copy(rows_a, o_ref.at[pl.ds(base + c0 * _SC_S, _SC_S)])

            @pl.when(k < nhalf - 1)
            def _():
                stage(c0 + 2, idx_a, rows_a, sem_a)

            drain(rows_b, sem_b)
            pltpu.sync_copy(
                rows_b, o_ref.at[pl.ds(base + (c0 + 1) * _SC_S, _SC_S)]
            )
            return carry

        lax.fori_loop(0, nhalf, body, 0)

    one(ta_ref, ia_ref, oa_ref)
    one(tb_ref, ib_ref, ob_ref)


def _sc_gather_pair(ta, ia, tb, ib):
    e = ia.shape[0]
    ep = _sc_pad(e)
    pad = ep - e
    ia_p = jnp.concatenate([ia, jnp.zeros((pad,), ia.dtype)])
    ib_p = jnp.concatenate([ib, jnp.zeros((pad,), ib.dtype)])
    run = pl.kernel(
        _sc_gather_pair_body,
        mesh=plsc.VectorSubcoreMesh(core_axis_name="c", subcore_axis_name="s"),
        out_type=(
            jax.ShapeDtypeStruct((ep, 128), jnp.float32),
            jax.ShapeDtypeStruct((ep, 128), jnp.float32),
        ),
        scratch_types=[
            pltpu.VMEM((_SC_S,), jnp.int32),
            pltpu.VMEM((_SC_S,), jnp.int32),
            pltpu.VMEM((_SC_S, 128), jnp.float32),
            pltpu.VMEM((_SC_S, 128), jnp.float32),
            pltpu.SemaphoreType.DMA,
            pltpu.SemaphoreType.DMA,
        ],
    )
    ga, gb = run(ta, ia_p, tb, ib_p)
    return ga[:e], gb[:e]


def _nmm_body(x_ref, w_ref, o_ref):
    o_ref[...] = _mm(x_ref[...], w_ref[...])


def _nmm(x, w):
    n, c = x.shape
    dout = w.shape[1]
    return _pcall(
        _nmm_body,
        grid=(n // NB,),
        in_specs=[pl.BlockSpec((NB, c), lambda i: (i, 0)), _full((c, dout))],
        out_specs=pl.BlockSpec((NB, dout), lambda i: (i, 0)),
        out_shape=jax.ShapeDtypeStruct((n, dout), jnp.float32),
    )(x, w)


def _dec_body(ps_ref, pd_ref, e_ref, w0c_ref, b0_ref, w1_ref, b1_ref,
              w2_ref, b2_ref, o_ref):
    h = ps_ref[...] + pd_ref[...] + _mm(e_ref[...], w0c_ref[...]) + b0_ref[...]
    h = jnp.maximum(h, 0.0)
    h = jnp.maximum(_mm(h, w1_ref[...]) + b1_ref[...], 0.0)
    o_ref[...] = _mm(h, w2_ref[...]) + b2_ref[...]


def _decoder(ps_g, pd_g, e1, p):
    ne = ps_g.shape[0]
    return _pcall(
        _dec_body,
        grid=(ne // EB,),
        in_specs=[
            pl.BlockSpec((EB, 128), lambda i: (i, 0)),
            pl.BlockSpec((EB, 128), lambda i: (i, 0)),
            pl.BlockSpec((EB, 128), lambda i: (i, 0)),
            _full((128, 128)), _full((1, 128)),
            _full((128, 128)), _full((1, 128)),
            _full((128, 1)), _full((1, 1)),
        ],
        out_specs=pl.BlockSpec((EB, 1), lambda i: (i, 0)),
        out_shape=jax.ShapeDtypeStruct((ne, 1), jnp.float32),
    )(ps_g, pd_g, e1, p["W0"][512:], p["b0"].reshape(1, 128),
      p["W1"], p["b1"].reshape(1, 128), p["W2"], p["b2"].reshape(1, 1))


def _gat_layer(x, src, dst, e, la, p, n):
    xl, xr, aloop = _proj(x, la, p)
    gxl, gxr = _sc_gather_pair(xl, src, xr, dst)
    alpha = _alpha(gxl, gxr, e, p["We"], p["att"])
    af = alpha[:, 0]
    amax = jnp.maximum(
        jax.ops.segment_max(af, dst, num_segments=n), aloop[:, 0]
    )
    w, ex = _weight(gxl, alpha, jnp.take(amax, dst)[:, None])
    ssum = jax.ops.segment_sum(w, dst, num_segments=n)
    dsum = jax.ops.segment_sum(ex[:, 0], dst, num_segments=n)
    exl = jnp.exp(aloop[:, 0] - amax)
    denom = dsum + exl + 1e-16
    return (ssum + exl[:, None] * xl) / denom[:, None] + p["bias"]


def _loop_attr(e, dst, n):
    ea_sum = jax.ops.segment_sum(e, dst, num_segments=n)
    cnt = jax.ops.segment_sum(
        jnp.ones((e.shape[0],), jnp.float32), dst, num_segments=n
    )
    return ea_sum / jnp.maximum(cnt, 1.0)[:, None]


def kernel(graph1_x, graph1_edge_index, graph1_edge_attr, graph1_batch,
           graph2_x, graph2_edge_index, graph2_edge_attr, t_value, params):
    n = graph1_x.shape[0]
    src1, dst1 = graph1_edge_index[0], graph1_edge_index[1]
    src2, dst2 = graph2_edge_index[0], graph2_edge_index[1]

    te = _mlp1(t_value[:, None], params["time_encoder"], t_value.shape[0])
    x = jnp.take(jnp.concatenate([te, te], axis=1), graph1_batch, axis=0)

    e1 = _mlp1(graph1_edge_attr[:, 0:1], params["encoder_edges"], EB)
    e2 = _mlp1(graph2_edge_attr[:, None], params["encoder_edges"], EB)

    la1 = _loop_attr(e1, dst1, n)
    la2 = _loop_attr(e2, dst2, n)

    for i in range(3):
        o1 = _gat_layer(x, src1, dst1, e1, la1, params["gnn_global"][i], n)
        o2 = _gat_layer(x, src2, dst2, e2, la2, params["gnn_filter"][i], n)
        x = jnp.concatenate([o1, o2], axis=1)

    dp = params["decoding_layer_edge"]
    ps = _nmm(x, dp["W0"][:256])
    pd = _nmm(x, dp["W0"][256:512])
    gps, gpd = _sc_gather_pair(ps, src1, pd, dst1)
    return _decoder(gps, gpd, e1, dp)


# speedup vs baseline: 1.2822x; 1.1300x over previous
import functools

import jax
import jax.numpy as jnp
from jax import lax
from jax.experimental import pallas as pl
from jax.experimental.pallas import tpu as pltpu
from jax.experimental.pallas import tpu_sc as plsc

_INTERPRET = False

EB = 1000
NB = 1000


def _mm(a, b):
    return jax.lax.dot_general(
        a, b, (((1,), (0,)), ((), ())), preferred_element_type=jnp.float32
    )


def _pcall(body, grid, in_specs, out_specs, out_shape):
    return pl.pallas_call(
        body,
        grid=grid,
        in_specs=in_specs,
        out_specs=out_specs,
        out_shape=out_shape,
        interpret=_INTERPRET,
    )


def _full(shape):
    return pl.BlockSpec(shape, lambda i: tuple(0 for _ in shape))


def _mlp1_body(ea_ref, w0_ref, b0_ref, w1_ref, b1_ref, w2_ref, b2_ref, o_ref):
    h = jnp.maximum(ea_ref[...] * w0_ref[...] + b0_ref[...], 0.0)
    h = jnp.maximum(_mm(h, w1_ref[...]) + b1_ref[...], 0.0)
    o_ref[...] = _mm(h, w2_ref[...]) + b2_ref[...]


def _mlp1(ea, p, block):
    e = ea.shape[0]
    dout = p["W2"].shape[1]
    return _pcall(
        _mlp1_body,
        grid=(e // block,),
        in_specs=[
            pl.BlockSpec((block, 1), lambda i: (i, 0)),
            _full((1, 128)), _full((1, 128)),
            _full((128, 128)), _full((1, 128)),
            _full((128, dout)), _full((1, dout)),
        ],
        out_specs=pl.BlockSpec((block, dout), lambda i: (i, 0)),
        out_shape=jax.ShapeDtypeStruct((e, dout), jnp.float32),
    )(ea, p["W0"].reshape(1, 128), p["b0"].reshape(1, 128),
      p["W1"], p["b1"].reshape(1, 128), p["W2"], p["b2"].reshape(1, -1))


def _proj_body(x_ref, la_ref, wl_ref, bl_ref, wr_ref, br_ref, we_ref, att_ref,
               xl_ref, xr_ref, al_ref):
    x = x_ref[...]
    xl = _mm(x, wl_ref[...]) + bl_ref[...]
    xr = _mm(x, wr_ref[...]) + br_ref[...]
    s = xl + xr + _mm(la_ref[...], we_ref[...])
    s = jnp.where(s > 0, s, 0.2 * s)
    xl_ref[...] = xl
    xr_ref[...] = xr
    al_ref[...] = jnp.sum(s * att_ref[...], axis=1, keepdims=True)


def _proj(x, la, p):
    n, c = x.shape
    out_shape = (
        jax.ShapeDtypeStruct((n, 128), jnp.float32),
        jax.ShapeDtypeStruct((n, 128), jnp.float32),
        jax.ShapeDtypeStruct((n, 1), jnp.float32),
    )
    return _pcall(
        _proj_body,
        grid=(n // NB,),
        in_specs=[
            pl.BlockSpec((NB, c), lambda i: (i, 0)),
            pl.BlockSpec((NB, 128), lambda i: (i, 0)),
            _full((c, 128)), _full((1, 128)),
            _full((c, 128)), _full((1, 128)),
            _full((128, 128)), _full((1, 128)),
        ],
        out_specs=(
            pl.BlockSpec((NB, 128), lambda i: (i, 0)),
            pl.BlockSpec((NB, 128), lambda i: (i, 0)),
            pl.BlockSpec((NB, 1), lambda i: (i, 0)),
        ),
        out_shape=out_shape,
    )(x, la, p["Wl"], p["bl"].reshape(1, 128), p["Wr"], p["br"].reshape(1, 128),
      p["We"], p["att"].reshape(1, 128))


def _alpha_body(gxl_ref, gxr_ref, e_ref, we_ref, att_ref, a_ref):
    s = gxl_ref[...] + gxr_ref[...] + _mm(e_ref[...], we_ref[...])
    s = jnp.where(s > 0, s, 0.2 * s)
    a_ref[...] = jnp.sum(s * att_ref[...], axis=1, keepdims=True)


def _alpha(gxl, gxr, e, we, att):
    ne = gxl.shape[0]
    return _pcall(
        _alpha_body,
        grid=(ne // EB,),
        in_specs=[
            pl.BlockSpec((EB, 128), lambda i: (i, 0)),
            pl.BlockSpec((EB, 128), lambda i: (i, 0)),
            pl.BlockSpec((EB, 128), lambda i: (i, 0)),
            _full((128, 128)), _full((1, 128)),
        ],
        out_specs=pl.BlockSpec((EB, 1), lambda i: (i, 0)),
        out_shape=jax.ShapeDtypeStruct((ne, 1), jnp.float32),
    )(gxl, gxr, e, we, att.reshape(1, 128))


def _weight_body(gxl_ref, a_ref, am_ref, w_ref, ex_ref):
    ex = jnp.exp(jnp.minimum(a_ref[...] - am_ref[...], 60.0))
    ex_ref[...] = ex
    w_ref[...] = gxl_ref[...] * ex


def _weight(gxl, alpha, amax_g):
    ne = gxl.shape[0]
    return _pcall(
        _weight_body,
        grid=(ne // EB,),
        in_specs=[
            pl.BlockSpec((EB, 128), lambda i: (i, 0)),
            pl.BlockSpec((EB, 1), lambda i: (i, 0)),
            pl.BlockSpec((EB, 1), lambda i: (i, 0)),
        ],
        out_specs=(
            pl.BlockSpec((EB, 128), lambda i: (i, 0)),
            pl.BlockSpec((EB, 1), lambda i: (i, 0)),
        ),
        out_shape=(
            jax.ShapeDtypeStruct((ne, 128), jnp.float32),
            jax.ShapeDtypeStruct((ne, 1), jnp.float32),
        ),
    )(gxl, alpha, amax_g)


_SC_C = 128
_SC_K = 2
_SC_S = _SC_C * _SC_K
_SC_NW = 32


def _sc_pad(e):
    m = 2 * _SC_S * _SC_NW
    return ((e + m - 1) // m) * m


def _sc_gather_pair_body(ta_ref, ia_ref, tb_ref, ib_ref, oa_ref, ob_ref,
                         idx_a, idx_b, rows_a, rows_b, sem_a, sem_b):
    wid = lax.axis_index("s") * 2 + lax.axis_index("c")
    per_w = ia_ref.shape[0] // _SC_NW

    def one(t_ref, i_ref, o_ref):
        base = wid * per_w
        nhalf = per_w // (2 * _SC_S)

        def fire(iv, rv, sem):
            for b in range(_SC_K):
                pltpu.async_copy(
                    t_ref.at[iv.at[pl.ds(b * _SC_C, _SC_C)]],
                    rv.at[pl.ds(b * _SC_C, _SC_C)],
                    sem,
                )

        def drain(rv, sem):
            pltpu.make_async_copy(t_ref.at[pl.ds(0, _SC_S)], rv, sem).wait()

        def stage(chunk, iv, rv, sem):
            pltpu.sync_copy(i_ref.at[pl.ds(base + chunk * _SC_S, _SC_S)], iv)
            fire(iv, rv, sem)

        stage(0, idx_a, rows_a, sem_a)

        def body(k, carry):
            c0 = 2 * k
            stage(c0 + 1, idx_b, rows_b, sem_b)
            drain(rows_a, sem_a)
            pltpu.sync_copy(rows_a, o_ref.at[pl.ds(base + c0 * _SC_S, _SC_S)])

            @pl.when(k < nhalf - 1)
            def _():
                stage(c0 + 2, idx_a, rows_a, sem_a)

            drain(rows_b, sem_b)
            pltpu.sync_copy(
                rows_b, o_ref.at[pl.ds(base + (c0 + 1) * _SC_S, _SC_S)]
            )
            return carry

        lax.fori_loop(0, nhalf, body, 0)

    one(ta_ref, ia_ref, oa_ref)
    one(tb_ref, ib_ref, ob_ref)


def _sc_gather_pair(ta, ia, tb, ib):
    e = ia.shape[0]
    ep = _sc_pad(e)
    pad = ep - e
    ia_p = jnp.concatenate([ia, jnp.zeros((pad,), ia.dtype)])
    ib_p = jnp.concatenate([ib, jnp.zeros((pad,), ib.dtype)])
    run = pl.kernel(
        _sc_gather_pair_body,
        mesh=plsc.VectorSubcoreMesh(core_axis_name="c", subcore_axis_name="s"),
        out_type=(
            jax.ShapeDtypeStruct((ep, 128), jnp.float32),
            jax.ShapeDtypeStruct((ep, 128), jnp.float32),
        ),
        scratch_types=[
            pltpu.VMEM((_SC_S,), jnp.int32),
            pltpu.VMEM((_SC_S,), jnp.int32),
            pltpu.VMEM((_SC_S, 128), jnp.float32),
            pltpu.VMEM((_SC_S, 128), jnp.float32),
            pltpu.SemaphoreType.DMA,
            pltpu.SemaphoreType.DMA,
        ],
    )
    ga, gb = run(ta, ia_p, tb, ib_p)
    return ga[:e], gb[:e]


def _nmm_body(x_ref, w_ref, o_ref):
    o_ref[...] = _mm(x_ref[...], w_ref[...])


def _nmm(x, w):
    n, c = x.shape
    dout = w.shape[1]
    return _pcall(
        _nmm_body,
        grid=(n // NB,),
        in_specs=[pl.BlockSpec((NB, c), lambda i: (i, 0)), _full((c, dout))],
        out_specs=pl.BlockSpec((NB, dout), lambda i: (i, 0)),
        out_shape=jax.ShapeDtypeStruct((n, dout), jnp.float32),
    )(x, w)


def _dec_body(ps_ref, pd_ref, e_ref, w0c_ref, b0_ref, w1_ref, b1_ref,
              w2_ref, b2_ref, o_ref):
    h = ps_ref[...] + pd_ref[...] + _mm(e_ref[...], w0c_ref[...]) + b0_ref[...]
    h = jnp.maximum(h, 0.0)
    h = jnp.maximum(_mm(h, w1_ref[...]) + b1_ref[...], 0.0)
    o_ref[...] = _mm(h, w2_ref[...]) + b2_ref[...]


def _decoder(ps_g, pd_g, e1, p):
    ne = ps_g.shape[0]
    return _pcall(
        _dec_body,
        grid=(ne // EB,),
        in_specs=[
            pl.BlockSpec((EB, 128), lambda i: (i, 0)),
            pl.BlockSpec((EB, 128), lambda i: (i, 0)),
            pl.BlockSpec((EB, 128), lambda i: (i, 0)),
            _full((128, 128)), _full((1, 128)),
            _full((128, 128)), _full((1, 128)),
            _full((128, 1)), _full((1, 1)),
        ],
        out_specs=pl.BlockSpec((EB, 1), lambda i: (i, 0)),
        out_shape=jax.ShapeDtypeStruct((ne, 1), jnp.float32),
    )(ps_g, pd_g, e1, p["W0"][512:], p["b0"].reshape(1, 128),
      p["W1"], p["b1"].reshape(1, 128), p["W2"], p["b2"].reshape(1, 1))


def _gat_layer(x, src, dst, e, la, p, n):
    xl, xr, aloop = _proj(x, la, p)
    gxl, gxr = _sc_gather_pair(xl, src, xr, dst)
    alpha = _alpha(gxl, gxr, e, p["We"], p["att"])
    w, ex = _weight(gxl, alpha, jnp.take(aloop[:, 0], dst)[:, None])
    ssum = jax.ops.segment_sum(w, dst, num_segments=n)
    dsum = jax.ops.segment_sum(ex[:, 0], dst, num_segments=n)
    denom = dsum + 1.0 + 1e-16
    return (ssum + xl) / denom[:, None] + p["bias"]


def _loop_attr(e, dst, n):
    ea_sum = jax.ops.segment_sum(e, dst, num_segments=n)
    cnt = jax.ops.segment_sum(
        jnp.ones((e.shape[0],), jnp.float32), dst, num_segments=n
    )
    return ea_sum / jnp.maximum(cnt, 1.0)[:, None]


def kernel(graph1_x, graph1_edge_index, graph1_edge_attr, graph1_batch,
           graph2_x, graph2_edge_index, graph2_edge_attr, t_value, params):
    n = graph1_x.shape[0]
    src1, dst1 = graph1_edge_index[0], graph1_edge_index[1]
    src2, dst2 = graph2_edge_index[0], graph2_edge_index[1]

    te = _mlp1(t_value[:, None], params["time_encoder"], t_value.shape[0])
    x = jnp.take(jnp.concatenate([te, te], axis=1), graph1_batch, axis=0)

    e1 = _mlp1(graph1_edge_attr[:, 0:1], params["encoder_edges"], EB)
    e2 = _mlp1(graph2_edge_attr[:, None], params["encoder_edges"], EB)

    la1 = _loop_attr(e1, dst1, n)
    la2 = _loop_attr(e2, dst2, n)

    for i in range(3):
        o1 = _gat_layer(x, src1, dst1, e1, la1, params["gnn_global"][i], n)
        o2 = _gat_layer(x, src2, dst2, e2, la2, params["gnn_filter"][i], n)
        x = jnp.concatenate([o1, o2], axis=1)

    dp = params["decoding_layer_edge"]
    ps = _nmm(x, dp["W0"][:256])
    pd = _nmm(x, dp["W0"][256:512])
    gps, gpd = _sc_gather_pair(ps, src1, pd, dst1)
    return _decoder(gps, gpd, e1, dp)


# speedup vs baseline: 1.2826x; 1.0002x over previous
import functools

import jax
import jax.numpy as jnp
from jax import lax
from jax.experimental import pallas as pl
from jax.experimental.pallas import tpu as pltpu
from jax.experimental.pallas import tpu_sc as plsc

EB = 1000
NB = 1000


def _mm(a, b):
    return jax.lax.dot_general(
        a, b, (((1,), (0,)), ((), ())), preferred_element_type=jnp.float32
    )


def _pcall(body, grid, in_specs, out_specs, out_shape):
    return pl.pallas_call(
        body,
        grid=grid,
        in_specs=in_specs,
        out_specs=out_specs,
        out_shape=out_shape,
    )


def _full(shape):
    return pl.BlockSpec(shape, lambda i: tuple(0 for _ in shape))


def _mlp1_body(ea_ref, w0_ref, b0_ref, w1_ref, b1_ref, w2_ref, b2_ref, o_ref):
    h = jnp.maximum(ea_ref[...] * w0_ref[...] + b0_ref[...], 0.0)
    h = jnp.maximum(_mm(h, w1_ref[...]) + b1_ref[...], 0.0)
    o_ref[...] = _mm(h, w2_ref[...]) + b2_ref[...]


def _mlp1(ea, p, block):
    e = ea.shape[0]
    dout = p["W2"].shape[1]
    return _pcall(
        _mlp1_body,
        grid=(e // block,),
        in_specs=[
            pl.BlockSpec((block, 1), lambda i: (i, 0)),
            _full((1, 128)), _full((1, 128)),
            _full((128, 128)), _full((1, 128)),
            _full((128, dout)), _full((1, dout)),
        ],
        out_specs=pl.BlockSpec((block, dout), lambda i: (i, 0)),
        out_shape=jax.ShapeDtypeStruct((e, dout), jnp.float32),
    )(ea, p["W0"].reshape(1, 128), p["b0"].reshape(1, 128),
      p["W1"], p["b1"].reshape(1, 128), p["W2"], p["b2"].reshape(1, -1))


def _proj_body(x_ref, la_ref, wl_ref, bl_ref, wr_ref, br_ref, we_ref, att_ref,
               xl_ref, xr_ref, al_ref):
    x = x_ref[...]
    xl = _mm(x, wl_ref[...]) + bl_ref[...]
    xr = _mm(x, wr_ref[...]) + br_ref[...]
    s = xl + xr + _mm(la_ref[...], we_ref[...])
    s = jnp.where(s > 0, s, 0.2 * s)
    xl_ref[...] = xl
    xr_ref[...] = xr
    al_ref[...] = jnp.sum(s * att_ref[...], axis=1, keepdims=True)


def _proj(x, la, p):
    n, c = x.shape
    out_shape = (
        jax.ShapeDtypeStruct((n, 128), jnp.float32),
        jax.ShapeDtypeStruct((n, 128), jnp.float32),
        jax.ShapeDtypeStruct((n, 1), jnp.float32),
    )
    return _pcall(
        _proj_body,
        grid=(n // NB,),
        in_specs=[
            pl.BlockSpec((NB, c), lambda i: (i, 0)),
            pl.BlockSpec((NB, 128), lambda i: (i, 0)),
            _full((c, 128)), _full((1, 128)),
            _full((c, 128)), _full((1, 128)),
            _full((128, 128)), _full((1, 128)),
        ],
        out_specs=(
            pl.BlockSpec((NB, 128), lambda i: (i, 0)),
            pl.BlockSpec((NB, 128), lambda i: (i, 0)),
            pl.BlockSpec((NB, 1), lambda i: (i, 0)),
        ),
        out_shape=out_shape,
    )(x, la, p["Wl"], p["bl"].reshape(1, 128), p["Wr"], p["br"].reshape(1, 128),
      p["We"], p["att"].reshape(1, 128))


def _alpha_body(gxl_ref, gxr_ref, e_ref, we_ref, att_ref, a_ref):
    s = gxl_ref[...] + gxr_ref[...] + _mm(e_ref[...], we_ref[...])
    s = jnp.where(s > 0, s, 0.2 * s)
    a_ref[...] = jnp.sum(s * att_ref[...], axis=1, keepdims=True)


def _alpha(gxl, gxr, e, we, att):
    ne = gxl.shape[0]
    return _pcall(
        _alpha_body,
        grid=(ne // EB,),
        in_specs=[
            pl.BlockSpec((EB, 128), lambda i: (i, 0)),
            pl.BlockSpec((EB, 128), lambda i: (i, 0)),
            pl.BlockSpec((EB, 128), lambda i: (i, 0)),
            _full((128, 128)), _full((1, 128)),
        ],
        out_specs=pl.BlockSpec((EB, 1), lambda i: (i, 0)),
        out_shape=jax.ShapeDtypeStruct((ne, 1), jnp.float32),
    )(gxl, gxr, e, we, att.reshape(1, 128))


def _weight_body(gxl_ref, a_ref, am_ref, w_ref, ex_ref):
    ex = jnp.exp(jnp.minimum(a_ref[...] - am_ref[...], 60.0))
    ex_ref[...] = ex
    w_ref[...] = gxl_ref[...] * ex


def _weight(gxl, alpha, amax_g):
    ne = gxl.shape[0]
    return _pcall(
        _weight_body,
        grid=(ne // EB,),
        in_specs=[
            pl.BlockSpec((EB, 128), lambda i: (i, 0)),
            pl.BlockSpec((EB, 1), lambda i: (i, 0)),
            pl.BlockSpec((EB, 1), lambda i: (i, 0)),
        ],
        out_specs=(
            pl.BlockSpec((EB, 128), lambda i: (i, 0)),
            pl.BlockSpec((EB, 1), lambda i: (i, 0)),
        ),
        out_shape=(
            jax.ShapeDtypeStruct((ne, 128), jnp.float32),
            jax.ShapeDtypeStruct((ne, 1), jnp.float32),
        ),
    )(gxl, alpha, amax_g)


_SC_C = 128
_SC_K = 2
_SC_S = _SC_C * _SC_K
_SC_NW = 32


def _sc_pad(e):
    m = 2 * _SC_S * _SC_NW
    return ((e + m - 1) // m) * m


def _sc_gather_pair_body(ta_ref, ia_ref, tb_ref, ib_ref, oa_ref, ob_ref,
                         idx_a, idx_b, rows_a, rows_b, sem_a, sem_b):
    wid = lax.axis_index("s") * 2 + lax.axis_index("c")
    per_w = ia_ref.shape[0] // _SC_NW

    def one(t_ref, i_ref, o_ref):
        base = wid * per_w
        nhalf = per_w // (2 * _SC_S)

        def fire(iv, rv, sem):
            for b in range(_SC_K):
                pltpu.async_copy(
                    t_ref.at[iv.at[pl.ds(b * _SC_C, _SC_C)]],
                    rv.at[pl.ds(b * _SC_C, _SC_C)],
                    sem,
                )

        def drain(rv, sem):
            pltpu.make_async_copy(t_ref.at[pl.ds(0, _SC_S)], rv, sem).wait()

        def stage(chunk, iv, rv, sem):
            pltpu.sync_copy(i_ref.at[pl.ds(base + chunk * _SC_S, _SC_S)], iv)
            fire(iv, rv, sem)

        stage(0, idx_a, rows_a, sem_a)

        def body(k, carry):
            c0 = 2 * k
            stage(c0 + 1, idx_b, rows_b, sem_b)
            drain(rows_a, sem_a)
            pltpu.sync_copy(rows_a, o_ref.at[pl.ds(base + c0 * _SC_S, _SC_S)])

            @pl.when(k < nhalf - 1)
            def _():
                stage(c0 + 2, idx_a, rows_a, sem_a)

            drain(rows_b, sem_b)
            pltpu.sync_copy(
                rows_b, o_ref.at[pl.ds(base + (c0 + 1) * _SC_S, _SC_S)]
            )
            return carry

        lax.fori_loop(0, nhalf, body, 0)

    one(ta_ref, ia_ref, oa_ref)
    one(tb_ref, ib_ref, ob_ref)


def _sc_gather_pair(ta, ia, tb, ib):
    e = ia.shape[0]
    ep = _sc_pad(e)
    pad = ep - e
    ia_p = jnp.concatenate([ia, jnp.zeros((pad,), ia.dtype)])
    ib_p = jnp.concatenate([ib, jnp.zeros((pad,), ib.dtype)])
    run = pl.kernel(
        _sc_gather_pair_body,
        mesh=plsc.VectorSubcoreMesh(core_axis_name="c", subcore_axis_name="s"),
        out_type=(
            jax.ShapeDtypeStruct((ep, 128), jnp.float32),
            jax.ShapeDtypeStruct((ep, 128), jnp.float32),
        ),
        scratch_types=[
            pltpu.VMEM((_SC_S,), jnp.int32),
            pltpu.VMEM((_SC_S,), jnp.int32),
            pltpu.VMEM((_SC_S, 128), jnp.float32),
            pltpu.VMEM((_SC_S, 128), jnp.float32),
            pltpu.SemaphoreType.DMA,
            pltpu.SemaphoreType.DMA,
        ],
    )
    ga, gb = run(ta, ia_p, tb, ib_p)
    return ga[:e], gb[:e]


def _nmm_body(x_ref, w_ref, o_ref):
    o_ref[...] = _mm(x_ref[...], w_ref[...])


def _nmm(x, w):
    n, c = x.shape
    dout = w.shape[1]
    return _pcall(
        _nmm_body,
        grid=(n // NB,),
        in_specs=[pl.BlockSpec((NB, c), lambda i: (i, 0)), _full((c, dout))],
        out_specs=pl.BlockSpec((NB, dout), lambda i: (i, 0)),
        out_shape=jax.ShapeDtypeStruct((n, dout), jnp.float32),
    )(x, w)


def _dec_body(ps_ref, pd_ref, e_ref, w0c_ref, b0_ref, w1_ref, b1_ref,
              w2_ref, b2_ref, o_ref):
    h = ps_ref[...] + pd_ref[...] + _mm(e_ref[...], w0c_ref[...]) + b0_ref[...]
    h = jnp.maximum(h, 0.0)
    h = jnp.maximum(_mm(h, w1_ref[...]) + b1_ref[...], 0.0)
    o_ref[...] = _mm(h, w2_ref[...]) + b2_ref[...]


def _decoder(ps_g, pd_g, e1, p):
    ne = ps_g.shape[0]
    return _pcall(
        _dec_body,
        grid=(ne // EB,),
        in_specs=[
            pl.BlockSpec((EB, 128), lambda i: (i, 0)),
            pl.BlockSpec((EB, 128), lambda i: (i, 0)),
            pl.BlockSpec((EB, 128), lambda i: (i, 0)),
            _full((128, 128)), _full((1, 128)),
            _full((128, 128)), _full((1, 128)),
            _full((128, 1)), _full((1, 1)),
        ],
        out_specs=pl.BlockSpec((EB, 1), lambda i: (i, 0)),
        out_shape=jax.ShapeDtypeStruct((ne, 1), jnp.float32),
    )(ps_g, pd_g, e1, p["W0"][512:], p["b0"].reshape(1, 128),
      p["W1"], p["b1"].reshape(1, 128), p["W2"], p["b2"].reshape(1, 1))


def _gat_layer(x, src, dst, e, la, p, n):
    xl, xr, aloop = _proj(x, la, p)
    gxl, gxr = _sc_gather_pair(xl, src, xr, dst)
    alpha = _alpha(gxl, gxr, e, p["We"], p["att"])
    w, ex = _weight(gxl, alpha, jnp.take(aloop[:, 0], dst)[:, None])
    ssum = jax.ops.segment_sum(w, dst, num_segments=n)
    dsum = jax.ops.segment_sum(ex[:, 0], dst, num_segments=n)
    denom = dsum + 1.0 + 1e-16
    return (ssum + xl) / denom[:, None] + p["bias"]


def _loop_attr(e, dst, n):
    ea_sum = jax.ops.segment_sum(e, dst, num_segments=n)
    cnt = jax.ops.segment_sum(
        jnp.ones((e.shape[0],), jnp.float32), dst, num_segments=n
    )
    return ea_sum / jnp.maximum(cnt, 1.0)[:, None]


def kernel(graph1_x, graph1_edge_index, graph1_edge_attr, graph1_batch,
           graph2_x, graph2_edge_index, graph2_edge_attr, t_value, params):
    n = graph1_x.shape[0]
    src1, dst1 = graph1_edge_index[0], graph1_edge_index[1]
    src2, dst2 = graph2_edge_index[0], graph2_edge_index[1]

    te = _mlp1(t_value[:, None], params["time_encoder"], t_value.shape[0])
    x = jnp.take(jnp.concatenate([te, te], axis=1), graph1_batch, axis=0)

    e1 = _mlp1(graph1_edge_attr[:, 0:1], params["encoder_edges"], EB)
    e2 = _mlp1(graph2_edge_attr[:, None], params["encoder_edges"], EB)

    la1 = _loop_attr(e1, dst1, n)
    la2 = _loop_attr(e2, dst2, n)

    for i in range(3):
        o1 = _gat_layer(x, src1, dst1, e1, la1, params["gnn_global"][i], n)
        o2 = _gat_layer(x, src2, dst2, e2, la2, params["gnn_filter"][i], n)
        x = jnp.concatenate([o1, o2], axis=1)

    dp = params["decoding_layer_edge"]
    ps = _nmm(x, dp["W0"][:256])
    pd = _nmm(x, dp["W0"][256:512])
    gps, gpd = _sc_gather_pair(ps, src1, pd, dst1)
    return _decoder(gps, gpd, e1, dp)
